# baseline (speedup 1.0000x reference)
import jax
import jax.numpy as jnp
from jax import lax
from jax.experimental import pallas as pl
from jax.experimental.pallas import tpu as pltpu

F0 = 32
H = 64
K = 128
EMB = 64
BB = 128
NL = BB * EMB


def _cin_body(xt_ref, wt0_hbm, wt1_hbm, wt2_hbm, g_hbm,
              b0_ref, b1_ref, b2_ref, o_ref,
              z_ref, wt0_v, wt1_v, wt2_v, g_v, sems):

    @pl.when(pl.program_id(0) == 0)
    def _load_consts():
        pltpu.make_async_copy(wt0_hbm, wt0_v, sems.at[0]).start()
        pltpu.make_async_copy(wt1_hbm, wt1_v, sems.at[1]).start()
        pltpu.make_async_copy(wt2_hbm, wt2_v, sems.at[2]).start()
        pltpu.make_async_copy(g_hbm, g_v, sems.at[3]).start()
        pltpu.make_async_copy(wt0_hbm, wt0_v, sems.at[0]).wait()
        pltpu.make_async_copy(wt1_hbm, wt1_v, sems.at[1]).wait()
        pltpu.make_async_copy(wt2_hbm, wt2_v, sems.at[2]).wait()
        pltpu.make_async_copy(g_hbm, g_v, sems.at[3]).wait()

    xt = xt_ref[...]
    reps = NL // 128
    bias0 = pltpu.repeat(b0_ref[...], reps, axis=1)
    bias1 = pltpu.repeat(b1_ref[...], reps, axis=1)
    bias2 = pltpu.repeat(b2_ref[...], reps, axis=1)

    for i in range(F0):
        z_ref[i * F0:(i + 1) * F0, :] = (
            jnp.broadcast_to(xt[i:i + 1, :], (F0, NL)) * xt)
    del i
    cur = jnp.dot(wt0_v[...], z_ref[0:F0 * F0, :],
                  preferred_element_type=jnp.float32) + bias0
    d0 = cur[H:, :]
    h = cur[:H, :].astype(jnp.bfloat16)

    for i in range(F0):
        z_ref[i * H:(i + 1) * H, :] = (
            jnp.broadcast_to(xt[i:i + 1, :], (H, NL)) * h)
    cur = jnp.dot(wt1_v[...], z_ref[...],
                  preferred_element_type=jnp.float32) + bias1
    d1 = cur[H:, :]
    h = cur[:H, :].astype(jnp.bfloat16)

    for i in range(F0):
        z_ref[i * H:(i + 1) * H, :] = (
            jnp.broadcast_to(xt[i:i + 1, :], (H, NL)) * h)
    cur = jnp.dot(wt2_v[...], z_ref[...],
                  preferred_element_type=jnp.float32) + bias2

    direct = jnp.concatenate([d0, d1, cur], axis=0)
    o_ref[0] = lax.dot_general(
        g_v[...], direct, (((0,), (1,)), ((), ())),
        preferred_element_type=jnp.float32)


def kernel(x, W0, b0, W1, b1, W2, b2):
    B = x.shape[0]
    nblk = B // BB

    xt = jnp.transpose(x, (1, 0, 2)).reshape(F0, B * EMB).astype(jnp.bfloat16)
    wt0 = W0.T.astype(jnp.bfloat16)
    wt1 = W1.T.astype(jnp.bfloat16)
    wt2 = W2.T.astype(jnp.bfloat16)
    b0t = jnp.broadcast_to(b0[:, None], (K, 128))
    b1t = jnp.broadcast_to(b1[:, None], (K, 128))
    b2t = jnp.broadcast_to(b2[:, None], (K, 128))
    g = jnp.kron(jnp.eye(BB, dtype=jnp.float32),
                 jnp.ones((EMB, 1), dtype=jnp.float32))

    out3 = pl.pallas_call(
        _cin_body,
        grid=(nblk,),
        in_specs=[
            pl.BlockSpec((F0, NL), lambda n: (0, n)),
            pl.BlockSpec(memory_space=pl.ANY),
            pl.BlockSpec(memory_space=pl.ANY),
            pl.BlockSpec(memory_space=pl.ANY),
            pl.BlockSpec(memory_space=pl.ANY),
            pl.BlockSpec((K, 128), lambda n: (0, 0)),
            pl.BlockSpec((K, 128), lambda n: (0, 0)),
            pl.BlockSpec((K, 128), lambda n: (0, 0)),
        ],
        out_specs=pl.BlockSpec((1, BB, 2 * K), lambda n: (n, 0, 0)),
        out_shape=jax.ShapeDtypeStruct((nblk, BB, 2 * K), jnp.float32),
        scratch_shapes=[
            pltpu.VMEM((F0 * H, NL), jnp.bfloat16),
            pltpu.VMEM((K, F0 * F0), jnp.bfloat16),
            pltpu.VMEM((K, F0 * H), jnp.bfloat16),
            pltpu.VMEM((K, F0 * H), jnp.bfloat16),
            pltpu.VMEM((NL, BB), jnp.float32),
            pltpu.SemaphoreType.DMA((4,)),
        ],
        compiler_params=pltpu.CompilerParams(
            dimension_semantics=("parallel",),
            vmem_limit_bytes=56 * 1024 * 1024,
        ),
    )(xt, wt0, wt1, wt2, g, b0t, b1t, b2t)

    return out3.reshape(B, 2 * K)


# speedup vs baseline: 1.0104x; 1.0104x over previous
import jax
import jax.numpy as jnp
from jax import lax
from jax.experimental import pallas as pl
from jax.experimental.pallas import tpu as pltpu

F0 = 32
H = 64
K = 128
EMB = 64
BB = 64
NL = BB * EMB


def _cin_body(xt_ref, wt0_hbm, wt1_hbm, wt2_hbm, g_hbm,
              b0_ref, b1_ref, b2_ref, o_ref,
              z_ref, wt0_v, wt1_v, wt2_v, g_v, sems):

    @pl.when(pl.program_id(0) == 0)
    def _load_consts():
        pltpu.make_async_copy(wt0_hbm, wt0_v, sems.at[0]).start()
        pltpu.make_async_copy(wt1_hbm, wt1_v, sems.at[1]).start()
        pltpu.make_async_copy(wt2_hbm, wt2_v, sems.at[2]).start()
        pltpu.make_async_copy(g_hbm, g_v, sems.at[3]).start()
        pltpu.make_async_copy(wt0_hbm, wt0_v, sems.at[0]).wait()
        pltpu.make_async_copy(wt1_hbm, wt1_v, sems.at[1]).wait()
        pltpu.make_async_copy(wt2_hbm, wt2_v, sems.at[2]).wait()
        pltpu.make_async_copy(g_hbm, g_v, sems.at[3]).wait()

    xt = xt_ref[...]
    reps = NL // 128
    bias0 = pltpu.repeat(b0_ref[...], reps, axis=1)
    bias1 = pltpu.repeat(b1_ref[...], reps, axis=1)
    bias2 = pltpu.repeat(b2_ref[...], reps, axis=1)

    for i in range(F0):
        z_ref[i * F0:(i + 1) * F0, :] = (
            jnp.broadcast_to(xt[i:i + 1, :], (F0, NL)) * xt)
    del i
    cur = jnp.dot(wt0_v[...], z_ref[0:F0 * F0, :],
                  preferred_element_type=jnp.float32) + bias0
    d0 = cur[H:, :]
    h = cur[:H, :].astype(jnp.bfloat16)

    for i in range(F0):
        z_ref[i * H:(i + 1) * H, :] = (
            jnp.broadcast_to(xt[i:i + 1, :], (H, NL)) * h)
    cur = jnp.dot(wt1_v[...], z_ref[...],
                  preferred_element_type=jnp.float32) + bias1
    d1 = cur[H:, :]
    h = cur[:H, :].astype(jnp.bfloat16)

    for i in range(F0):
        z_ref[i * H:(i + 1) * H, :] = (
            jnp.broadcast_to(xt[i:i + 1, :], (H, NL)) * h)
    cur = jnp.dot(wt2_v[...], z_ref[...],
                  preferred_element_type=jnp.float32) + bias2

    direct = jnp.concatenate([d0, d1, cur], axis=0)
    o_ref[0] = lax.dot_general(
        g_v[...], direct, (((0,), (1,)), ((), ())),
        preferred_element_type=jnp.float32)


def kernel(x, W0, b0, W1, b1, W2, b2):
    B = x.shape[0]
    nblk = B // BB

    xt = jnp.transpose(x, (1, 0, 2)).reshape(F0, B * EMB).astype(jnp.bfloat16)
    wt0 = W0.T.astype(jnp.bfloat16)
    wt1 = W1.T.astype(jnp.bfloat16)
    wt2 = W2.T.astype(jnp.bfloat16)
    b0t = jnp.broadcast_to(b0[:, None], (K, 128))
    b1t = jnp.broadcast_to(b1[:, None], (K, 128))
    b2t = jnp.broadcast_to(b2[:, None], (K, 128))
    g = jnp.kron(jnp.eye(BB, dtype=jnp.float32),
                 jnp.ones((EMB, 1), dtype=jnp.float32))

    out3 = pl.pallas_call(
        _cin_body,
        grid=(nblk,),
        in_specs=[
            pl.BlockSpec((F0, NL), lambda n: (0, n)),
            pl.BlockSpec(memory_space=pl.ANY),
            pl.BlockSpec(memory_space=pl.ANY),
            pl.BlockSpec(memory_space=pl.ANY),
            pl.BlockSpec(memory_space=pl.ANY),
            pl.BlockSpec((K, 128), lambda n: (0, 0)),
            pl.BlockSpec((K, 128), lambda n: (0, 0)),
            pl.BlockSpec((K, 128), lambda n: (0, 0)),
        ],
        out_specs=pl.BlockSpec((1, BB, 2 * K), lambda n: (n, 0, 0)),
        out_shape=jax.ShapeDtypeStruct((nblk, BB, 2 * K), jnp.float32),
        scratch_shapes=[
            pltpu.VMEM((F0 * H, NL), jnp.bfloat16),
            pltpu.VMEM((K, F0 * F0), jnp.bfloat16),
            pltpu.VMEM((K, F0 * H), jnp.bfloat16),
            pltpu.VMEM((K, F0 * H), jnp.bfloat16),
            pltpu.VMEM((NL, BB), jnp.float32),
            pltpu.SemaphoreType.DMA((4,)),
        ],
        compiler_params=pltpu.CompilerParams(
            dimension_semantics=("parallel",),
            vmem_limit_bytes=56 * 1024 * 1024,
        ),
    )(xt, wt0, wt1, wt2, g, b0t, b1t, b2t)

    return out3.reshape(B, 2 * K)
